# baseline (speedup 1.0000x reference)
import functools

import jax
import jax.numpy as jnp
from jax import lax
from jax.experimental import pallas as pl
from jax.experimental.pallas import tpu as pltpu
from jax.experimental.pallas import tpu_sc as plsc

N = 10000
D = 128
NP = 10112
NC = 2
NS = 16
NW = NC * NS
CH = 128
E = 320000
SB = 16
NCH = 80
TCH = NCH * NW
EPT = NCH * CH
EP = EPT * NW
NSB = NCH // SB
STR = NP // NS
DW = 16
RB = 1000


def _deg_body(dstp_hbm, ones_hbm, zero_hbm, out_hbm, idx_d, ones_v, acc):
    c = lax.axis_index("c")
    s = lax.axis_index("s")
    t = c * NS + s
    pltpu.sync_copy(dstp_hbm.at[t], idx_d)
    pltpu.sync_copy(ones_hbm, ones_v)
    pltpu.sync_copy(zero_hbm.at[s], acc.at[pl.ds(s * STR, STR)])
    plsc.subcore_barrier()

    def body(ch, _):
        pltpu.sync_copy(ones_v, acc.at[idx_d.at[ch]], add=True)
        return ()

    lax.fori_loop(0, NCH, body, ())
    plsc.subcore_barrier()
    pltpu.sync_copy(acc.at[pl.ds(s * STR, STR)], out_hbm.at[c, s])


@functools.cache
def _deg_call_fn():
    return functools.partial(
        pl.kernel,
        out_type=jax.ShapeDtypeStruct((NC, NS, STR, DW), jnp.float32),
        mesh=plsc.VectorSubcoreMesh(core_axis_name="c", subcore_axis_name="s"),
        scratch_types=[
            pltpu.VMEM((NCH, CH), jnp.int32),
            pltpu.VMEM((CH, DW), jnp.float32),
            pltpu.VMEM_SHARED((NP, DW), jnp.float32),
        ],
        compiler_params=pltpu.CompilerParams(use_tc_tiling_on_sc=False),
    )(_deg_body)


def _deg_call(*args):
    return _deg_call_fn()(*args)


def _segsum_body(xs_hbm, srcp_hbm, dstp_hbm, zero_hbm, out_hbm,
                 idx_s, idx_d, rows0, rows1, sem0, sem1, ssem0, ssem1, acc):
    c = lax.axis_index("c")
    s = lax.axis_index("s")
    base = (c * NS + s) * NCH
    pltpu.sync_copy(zero_hbm.at[pl.ds(s * STR, STR)],
                    acc.at[pl.ds(s * STR, STR)])
    plsc.subcore_barrier()

    def sb_body(sb, _):
        pltpu.sync_copy(srcp_hbm.at[pl.ds(base + sb * SB, SB)], idx_s)
        pltpu.sync_copy(dstp_hbm.at[pl.ds(base + sb * SB, SB)], idx_d)
        pltpu.async_copy(xs_hbm.at[idx_s.at[0]], rows0, sem0)
        pltpu.async_copy(xs_hbm.at[idx_s.at[1]], rows1, sem1)

        def pair_body(i, _):
            ch = i * 2
            pltpu.make_async_copy(xs_hbm.at[idx_s.at[ch]], rows0,
                                  sem0).wait()
            pltpu.async_copy(rows0, acc.at[idx_d.at[ch]], ssem0, add=True)
            pltpu.make_async_copy(xs_hbm.at[idx_s.at[ch + 1]], rows1,
                                  sem1).wait()
            pltpu.async_copy(rows1, acc.at[idx_d.at[ch + 1]], ssem1, add=True)
            pltpu.make_async_copy(rows0, acc.at[idx_d.at[ch]], ssem0).wait()
            pltpu.async_copy(xs_hbm.at[idx_s.at[ch + 2]], rows0, sem0)
            pltpu.make_async_copy(rows1, acc.at[idx_d.at[ch + 1]],
                                  ssem1).wait()
            pltpu.async_copy(xs_hbm.at[idx_s.at[ch + 3]], rows1, sem1)
            return ()

        lax.fori_loop(0, SB // 2 - 1, pair_body, ())
        pltpu.make_async_copy(xs_hbm.at[idx_s.at[SB - 2]], rows0,
                              sem0).wait()
        pltpu.async_copy(rows0, acc.at[idx_d.at[SB - 2]], ssem0, add=True)
        pltpu.make_async_copy(xs_hbm.at[idx_s.at[SB - 1]], rows1,
                              sem1).wait()
        pltpu.async_copy(rows1, acc.at[idx_d.at[SB - 1]], ssem1, add=True)
        pltpu.make_async_copy(rows0, acc.at[idx_d.at[SB - 2]], ssem0).wait()
        pltpu.make_async_copy(rows1, acc.at[idx_d.at[SB - 1]], ssem1).wait()
        return ()

    lax.fori_loop(0, NSB, sb_body, ())
    plsc.subcore_barrier()
    pltpu.sync_copy(acc.at[pl.ds(s * STR, STR)],
                    out_hbm.at[c, pl.ds(s * STR, STR)])


@functools.cache
def _segsum_call_fn():
    return functools.partial(
        pl.kernel,
        out_type=jax.ShapeDtypeStruct((NC, NP, D), jnp.float32),
        mesh=plsc.VectorSubcoreMesh(core_axis_name="c", subcore_axis_name="s"),
        scratch_types=[
            pltpu.VMEM((SB, CH), jnp.int32),
            pltpu.VMEM((SB, CH), jnp.int32),
            pltpu.VMEM((CH, D), jnp.float32),
            pltpu.VMEM((CH, D), jnp.float32),
            pltpu.SemaphoreType.DMA,
            pltpu.SemaphoreType.DMA,
            pltpu.SemaphoreType.DMA,
            pltpu.SemaphoreType.DMA,
            pltpu.VMEM_SHARED((NP, D), jnp.float32),
        ],
    )(_segsum_body)


def _segsum_call(*args):
    return _segsum_call_fn()(*args)


def _tc1a_body(rep_ref, w1_ref, xw_ref):
    xw_ref[...] = jnp.dot(rep_ref[...], w1_ref[...],
                          preferred_element_type=jnp.float32)


def _tc1a(rep, w1):
    return pl.pallas_call(
        _tc1a_body,
        grid=(N // RB,),
        in_specs=[
            pl.BlockSpec((RB, D), lambda i: (i, 0)),
            pl.BlockSpec((D, D), lambda i: (0, 0)),
        ],
        out_specs=pl.BlockSpec((RB, D), lambda i: (i, 0)),
        out_shape=jax.ShapeDtypeStruct((N, D), jnp.float32),
    )(rep, w1)


def _tc1b_body(degp_ref, xw_ref, xs_ref, dinv_ref):
    cnt = (degp_ref[0] + degp_ref[1])[:, 0:1]
    dinv = lax.rsqrt(cnt + 2.0)
    xs_ref[...] = xw_ref[...] * dinv
    dinv_ref[...] = jnp.broadcast_to(dinv, (RB, D))


def _tc1b(degp, xw):
    return pl.pallas_call(
        _tc1b_body,
        grid=(N // RB,),
        in_specs=[
            pl.BlockSpec((NC, RB, DW), lambda i: (0, i, 0)),
            pl.BlockSpec((RB, D), lambda i: (i, 0)),
        ],
        out_specs=[
            pl.BlockSpec((RB, D), lambda i: (i, 0)),
            pl.BlockSpec((RB, D), lambda i: (i, 0)),
        ],
        out_shape=[
            jax.ShapeDtypeStruct((N, D), jnp.float32),
            jax.ShapeDtypeStruct((N, D), jnp.float32),
        ],
    )(degp, xw)


def _tc2_body(accp_ref, xs_ref, dinv_ref, b_ref, w_ref, xs2_ref):
    acc = accp_ref[0] + accp_ref[1]
    dinv = dinv_ref[...]
    h = jnp.maximum(dinv * (acc + 2.0 * xs_ref[...]) + b_ref[...][None, :], 0.0)
    xs2_ref[...] = jnp.dot(h, w_ref[...],
                           preferred_element_type=jnp.float32) * dinv


def _tc2(accp, xs, dinv, b, w):
    return pl.pallas_call(
        _tc2_body,
        grid=(N // RB,),
        in_specs=[
            pl.BlockSpec((NC, RB, D), lambda i: (0, i, 0)),
            pl.BlockSpec((RB, D), lambda i: (i, 0)),
            pl.BlockSpec((RB, D), lambda i: (i, 0)),
            pl.BlockSpec((D,), lambda i: (0,)),
            pl.BlockSpec((D, D), lambda i: (0, 0)),
        ],
        out_specs=pl.BlockSpec((RB, D), lambda i: (i, 0)),
        out_shape=jax.ShapeDtypeStruct((N, D), jnp.float32),
    )(accp, xs, dinv, b, w)


def _tc3_body(accp_ref, xs2_ref, dinv_ref, b2_ref, rep_ref, wl1a_ref,
              wl1b_ref, bl1_ref, wl2_ref, bl2_ref, u_ref,
              prob_ref, samp_ref):
    acc = accp_ref[0] + accp_ref[1]
    dinv = dinv_ref[...]
    h2 = jnp.maximum(dinv * (acc + 2.0 * xs2_ref[...]) + b2_ref[...][None, :],
                     0.0)
    z = jnp.dot(rep_ref[...], wl1a_ref[...], preferred_element_type=jnp.float32)
    z = z + jnp.dot(h2, wl1b_ref[...], preferred_element_type=jnp.float32)
    z = jnp.maximum(z + bl1_ref[...][None, :], 0.0)
    logit = jnp.dot(z, wl2_ref[...],
                    preferred_element_type=jnp.float32) + bl2_ref[0, 0]
    p = 1.0 / (1.0 + jnp.exp(-logit))
    u0 = u_ref[:, 0:1]
    u1 = u_ref[:, 1:2]
    g0 = -jnp.log(-jnp.log(u0))
    g1 = -jnp.log(-jnp.log(u1))
    prob_ref[...] = p
    samp_ref[...] = jnp.where(p + g1 > 1.0 - p + g0, 1.0, 0.0)


def _tc3(accp, xs2, dinv, b2, rep, wl1a, wl1b, bl1, wl2, bl2, u):
    return pl.pallas_call(
        _tc3_body,
        grid=(N // RB,),
        in_specs=[
            pl.BlockSpec((NC, RB, D), lambda i: (0, i, 0)),
            pl.BlockSpec((RB, D), lambda i: (i, 0)),
            pl.BlockSpec((RB, D), lambda i: (i, 0)),
            pl.BlockSpec((D,), lambda i: (0,)),
            pl.BlockSpec((RB, D), lambda i: (i, 0)),
            pl.BlockSpec((D, 64), lambda i: (0, 0)),
            pl.BlockSpec((D, 64), lambda i: (0, 0)),
            pl.BlockSpec((64,), lambda i: (0,)),
            pl.BlockSpec((64, 1), lambda i: (0, 0)),
            pl.BlockSpec((1, 1), lambda i: (0, 0)),
            pl.BlockSpec((RB, 2), lambda i: (i, 0)),
        ],
        out_specs=[
            pl.BlockSpec((RB, 1), lambda i: (i, 0)),
            pl.BlockSpec((RB, 1), lambda i: (i, 0)),
        ],
        out_shape=[
            jax.ShapeDtypeStruct((N, 1), jnp.float32),
            jax.ShapeDtypeStruct((N, 1), jnp.float32),
        ],
    )(accp, xs2, dinv, b2, rep, wl1a, wl1b, bl1, wl2, bl2, u)


def kernel(rep, edge_index, W1, b1, W2, b2, Wl1, bl1, Wl2, bl2):
    src = edge_index[0]
    dst = edge_index[1]
    pad = EP - E
    pad_idx = jnp.arange(pad, dtype=jnp.int32)
    srcp = jnp.concatenate([src, (pad_idx * 977) % N]).reshape(TCH, CH)
    dstp = jnp.concatenate([dst, N + pad_idx % (NP - N)]).reshape(TCH, CH)

    ones_dw = jnp.ones((CH, DW), jnp.float32)
    zeros_dw = jnp.zeros((NS, STR, DW), jnp.float32)
    zeros_d = jnp.zeros((NP, D), jnp.float32)

    degp = _deg_call(dstp.reshape(NW, NCH, CH), ones_dw,
                     zeros_dw).reshape(NC, NP, DW)
    xw1 = _tc1a(rep, W1)
    xs1, dinv = _tc1b(degp, xw1)
    acc1p = _segsum_call(xs1, srcp, dstp, zeros_d)
    xs2 = _tc2(acc1p, xs1, dinv, b1, W2)
    acc2p = _segsum_call(xs2, srcp, dstp, zeros_d)

    u = jax.random.uniform(jax.random.key(1234), (N, 2),
                           minval=1e-8, maxval=1.0)
    wl1a = Wl1[:D]
    wl1b = Wl1[D:]
    prob, samp = _tc3(acc2p, xs2, dinv, b2, rep, wl1a, wl1b, bl1, Wl2,
                      bl2.reshape(1, 1), u)
    return (prob.reshape(N), samp.reshape(N))


# speedup vs baseline: 1.0687x; 1.0687x over previous
import functools

import jax
import jax.numpy as jnp
from jax import lax
from jax.experimental import pallas as pl
from jax.experimental.pallas import tpu as pltpu
from jax.experimental.pallas import tpu_sc as plsc

N = 10000
D = 128
NP = 10112
NC = 2
NS = 16
NW = NC * NS
CH = 128
E = 320000
SB = 16
NCH = 80
TCH = NCH * NW
EPT = NCH * CH
EP = EPT * NW
NSB = NCH // SB
STR = NP // NS
DW = 16
RB = 1000


def _deg_body(dstp_hbm, ones_hbm, zero_hbm, out_hbm, idx_d, ones_v, acc):
    c = lax.axis_index("c")
    s = lax.axis_index("s")
    t = c * NS + s
    pltpu.sync_copy(dstp_hbm.at[t], idx_d)
    pltpu.sync_copy(ones_hbm, ones_v)
    pltpu.sync_copy(zero_hbm.at[s], acc.at[pl.ds(s * STR, STR)])
    plsc.subcore_barrier()

    def body(ch, _):
        pltpu.sync_copy(ones_v, acc.at[idx_d.at[ch]], add=True)
        return ()

    lax.fori_loop(0, NCH, body, ())
    plsc.subcore_barrier()
    pltpu.sync_copy(acc.at[pl.ds(s * STR, STR)], out_hbm.at[c, s])


@functools.cache
def _deg_call_fn():
    return functools.partial(
        pl.kernel,
        out_type=jax.ShapeDtypeStruct((NC, NS, STR, DW), jnp.float32),
        mesh=plsc.VectorSubcoreMesh(core_axis_name="c", subcore_axis_name="s"),
        scratch_types=[
            pltpu.VMEM((NCH, CH), jnp.int32),
            pltpu.VMEM((CH, DW), jnp.float32),
            pltpu.VMEM_SHARED((NP, DW), jnp.float32),
        ],
        compiler_params=pltpu.CompilerParams(use_tc_tiling_on_sc=False),
    )(_deg_body)


def _deg_call(*args):
    return _deg_call_fn()(*args)


def _segsum_body(xs_hbm, srcp_hbm, dstp_hbm, zero_hbm, out_hbm,
                 idx_s, idx_d, rows0, rows1, sem0, sem1, acc):
    c = lax.axis_index("c")
    s = lax.axis_index("s")
    base = (c * NS + s) * NCH
    pltpu.sync_copy(zero_hbm.at[pl.ds(s * STR, STR)],
                    acc.at[pl.ds(s * STR, STR)])
    plsc.subcore_barrier()

    def sb_body(sb, _):
        pltpu.sync_copy(srcp_hbm.at[pl.ds(base + sb * SB, SB)], idx_s)
        pltpu.sync_copy(dstp_hbm.at[pl.ds(base + sb * SB, SB)], idx_d)
        pltpu.async_copy(xs_hbm.at[idx_s.at[0]], rows0, sem0)

        def pair_body(i, _):
            ch = i * 2
            pltpu.make_async_copy(xs_hbm.at[idx_s.at[ch]], rows0,
                                  sem0).wait()
            pltpu.async_copy(xs_hbm.at[idx_s.at[ch + 1]], rows1, sem1)
            pltpu.sync_copy(rows0, acc.at[idx_d.at[ch]], add=True)

            pltpu.make_async_copy(xs_hbm.at[idx_s.at[ch + 1]], rows1,
                                  sem1).wait()
            pltpu.async_copy(xs_hbm.at[idx_s.at[ch + 2]], rows0, sem0)
            pltpu.sync_copy(rows1, acc.at[idx_d.at[ch + 1]], add=True)
            return ()

        lax.fori_loop(0, SB // 2 - 1, pair_body, ())
        pltpu.make_async_copy(xs_hbm.at[idx_s.at[SB - 2]], rows0,
                              sem0).wait()
        pltpu.async_copy(xs_hbm.at[idx_s.at[SB - 1]], rows1, sem1)
        pltpu.sync_copy(rows0, acc.at[idx_d.at[SB - 2]], add=True)
        pltpu.make_async_copy(xs_hbm.at[idx_s.at[SB - 1]], rows1,
                              sem1).wait()
        pltpu.sync_copy(rows1, acc.at[idx_d.at[SB - 1]], add=True)
        return ()

    lax.fori_loop(0, NSB, sb_body, ())
    plsc.subcore_barrier()
    pltpu.sync_copy(acc.at[pl.ds(s * STR, STR)],
                    out_hbm.at[c, pl.ds(s * STR, STR)])


@functools.cache
def _segsum_call_fn():
    return functools.partial(
        pl.kernel,
        out_type=jax.ShapeDtypeStruct((NC, NP, D), jnp.float32),
        mesh=plsc.VectorSubcoreMesh(core_axis_name="c", subcore_axis_name="s"),
        scratch_types=[
            pltpu.VMEM((SB, CH), jnp.int32),
            pltpu.VMEM((SB, CH), jnp.int32),
            pltpu.VMEM((CH, D), jnp.float32),
            pltpu.VMEM((CH, D), jnp.float32),
            pltpu.SemaphoreType.DMA,
            pltpu.SemaphoreType.DMA,
            pltpu.VMEM_SHARED((NP, D), jnp.float32),
        ],
    )(_segsum_body)


def _segsum_call(*args):
    return _segsum_call_fn()(*args)


def _tc1a_body(rep_ref, w1_ref, xw_ref):
    xw_ref[...] = jnp.dot(rep_ref[...], w1_ref[...],
                          preferred_element_type=jnp.float32)


def _tc1a(rep, w1):
    return pl.pallas_call(
        _tc1a_body,
        grid=(N // RB,),
        in_specs=[
            pl.BlockSpec((RB, D), lambda i: (i, 0)),
            pl.BlockSpec((D, D), lambda i: (0, 0)),
        ],
        out_specs=pl.BlockSpec((RB, D), lambda i: (i, 0)),
        out_shape=jax.ShapeDtypeStruct((N, D), jnp.float32),
    )(rep, w1)


def _tc1b_body(degp_ref, xw_ref, xs_ref, dinv_ref):
    cnt = (degp_ref[0] + degp_ref[1])[:, 0:1]
    dinv = lax.rsqrt(cnt + 2.0)
    xs_ref[...] = xw_ref[...] * dinv
    dinv_ref[...] = jnp.broadcast_to(dinv, (RB, D))


def _tc1b(degp, xw):
    return pl.pallas_call(
        _tc1b_body,
        grid=(N // RB,),
        in_specs=[
            pl.BlockSpec((NC, RB, DW), lambda i: (0, i, 0)),
            pl.BlockSpec((RB, D), lambda i: (i, 0)),
        ],
        out_specs=[
            pl.BlockSpec((RB, D), lambda i: (i, 0)),
            pl.BlockSpec((RB, D), lambda i: (i, 0)),
        ],
        out_shape=[
            jax.ShapeDtypeStruct((N, D), jnp.float32),
            jax.ShapeDtypeStruct((N, D), jnp.float32),
        ],
    )(degp, xw)


def _tc2_body(accp_ref, xs_ref, dinv_ref, b_ref, w_ref, xs2_ref):
    acc = accp_ref[0] + accp_ref[1]
    dinv = dinv_ref[...]
    h = jnp.maximum(dinv * (acc + 2.0 * xs_ref[...]) + b_ref[...][None, :], 0.0)
    xs2_ref[...] = jnp.dot(h, w_ref[...],
                           preferred_element_type=jnp.float32) * dinv


def _tc2(accp, xs, dinv, b, w):
    return pl.pallas_call(
        _tc2_body,
        grid=(N // RB,),
        in_specs=[
            pl.BlockSpec((NC, RB, D), lambda i: (0, i, 0)),
            pl.BlockSpec((RB, D), lambda i: (i, 0)),
            pl.BlockSpec((RB, D), lambda i: (i, 0)),
            pl.BlockSpec((D,), lambda i: (0,)),
            pl.BlockSpec((D, D), lambda i: (0, 0)),
        ],
        out_specs=pl.BlockSpec((RB, D), lambda i: (i, 0)),
        out_shape=jax.ShapeDtypeStruct((N, D), jnp.float32),
    )(accp, xs, dinv, b, w)


def _tc3_body(accp_ref, xs2_ref, dinv_ref, b2_ref, rep_ref, wl1a_ref,
              wl1b_ref, bl1_ref, wl2_ref, bl2_ref, u_ref,
              prob_ref, samp_ref):
    acc = accp_ref[0] + accp_ref[1]
    dinv = dinv_ref[...]
    h2 = jnp.maximum(dinv * (acc + 2.0 * xs2_ref[...]) + b2_ref[...][None, :],
                     0.0)
    z = jnp.dot(rep_ref[...], wl1a_ref[...], preferred_element_type=jnp.float32)
    z = z + jnp.dot(h2, wl1b_ref[...], preferred_element_type=jnp.float32)
    z = jnp.maximum(z + bl1_ref[...][None, :], 0.0)
    logit = jnp.dot(z, wl2_ref[...],
                    preferred_element_type=jnp.float32) + bl2_ref[0, 0]
    p = 1.0 / (1.0 + jnp.exp(-logit))
    u0 = u_ref[:, 0:1]
    u1 = u_ref[:, 1:2]
    g0 = -jnp.log(-jnp.log(u0))
    g1 = -jnp.log(-jnp.log(u1))
    prob_ref[...] = p
    samp_ref[...] = jnp.where(p + g1 > 1.0 - p + g0, 1.0, 0.0)


def _tc3(accp, xs2, dinv, b2, rep, wl1a, wl1b, bl1, wl2, bl2, u):
    return pl.pallas_call(
        _tc3_body,
        grid=(N // RB,),
        in_specs=[
            pl.BlockSpec((NC, RB, D), lambda i: (0, i, 0)),
            pl.BlockSpec((RB, D), lambda i: (i, 0)),
            pl.BlockSpec((RB, D), lambda i: (i, 0)),
            pl.BlockSpec((D,), lambda i: (0,)),
            pl.BlockSpec((RB, D), lambda i: (i, 0)),
            pl.BlockSpec((D, 64), lambda i: (0, 0)),
            pl.BlockSpec((D, 64), lambda i: (0, 0)),
            pl.BlockSpec((64,), lambda i: (0,)),
            pl.BlockSpec((64, 1), lambda i: (0, 0)),
            pl.BlockSpec((1, 1), lambda i: (0, 0)),
            pl.BlockSpec((RB, 2), lambda i: (i, 0)),
        ],
        out_specs=[
            pl.BlockSpec((RB, 1), lambda i: (i, 0)),
            pl.BlockSpec((RB, 1), lambda i: (i, 0)),
        ],
        out_shape=[
            jax.ShapeDtypeStruct((N, 1), jnp.float32),
            jax.ShapeDtypeStruct((N, 1), jnp.float32),
        ],
    )(accp, xs2, dinv, b2, rep, wl1a, wl1b, bl1, wl2, bl2, u)


def kernel(rep, edge_index, W1, b1, W2, b2, Wl1, bl1, Wl2, bl2):
    src = edge_index[0]
    dst = edge_index[1]
    pad = EP - E
    pad_idx = jnp.arange(pad, dtype=jnp.int32)
    srcp = jnp.concatenate([src, (pad_idx * 977) % N]).reshape(TCH, CH)
    dstp = jnp.concatenate([dst, N + pad_idx % (NP - N)]).reshape(TCH, CH)

    ones_dw = jnp.ones((CH, DW), jnp.float32)
    zeros_dw = jnp.zeros((NS, STR, DW), jnp.float32)
    zeros_d = jnp.zeros((NP, D), jnp.float32)

    degp = _deg_call(dstp.reshape(NW, NCH, CH), ones_dw,
                     zeros_dw).reshape(NC, NP, DW)
    xw1 = _tc1a(rep, W1)
    xs1, dinv = _tc1b(degp, xw1)
    acc1p = _segsum_call(xs1, srcp, dstp, zeros_d)
    xs2 = _tc2(acc1p, xs1, dinv, b1, W2)
    acc2p = _segsum_call(xs2, srcp, dstp, zeros_d)

    u = jax.random.uniform(jax.random.key(1234), (N, 2),
                           minval=1e-8, maxval=1.0)
    wl1a = Wl1[:D]
    wl1b = Wl1[D:]
    prob, samp = _tc3(acc2p, xs2, dinv, b2, rep, wl1a, wl1b, bl1, Wl2,
                      bl2.reshape(1, 1), u)
    return (prob.reshape(N), samp.reshape(N))


# speedup vs baseline: 1.0722x; 1.0033x over previous
import functools

import jax
import jax.numpy as jnp
from jax import lax
from jax.experimental import pallas as pl
from jax.experimental.pallas import tpu as pltpu
from jax.experimental.pallas import tpu_sc as plsc

N = 10000
D = 128
NP = 10112
NC = 2
NS = 16
NW = NC * NS
CH = 128
E = 320000
SB = 16
NCH = 80
TCH = NCH * NW
EPT = NCH * CH
EP = EPT * NW
NSB = NCH // SB
STR = NP // NS
DW = 16

import numpy as _np
_PAD_IDX = _np.arange(EP - E, dtype=_np.int32)
_PAD_SRC = _PAD_IDX * 977 % N
_PAD_DST = N + _PAD_IDX % (NP - N)
RB = 1000


def _deg_body(dstp_hbm, ones_hbm, zero_hbm, out_hbm, idx_d, ones_v, acc):
    c = lax.axis_index("c")
    s = lax.axis_index("s")
    t = c * NS + s
    pltpu.sync_copy(dstp_hbm.at[t], idx_d)
    pltpu.sync_copy(ones_hbm, ones_v)
    pltpu.sync_copy(zero_hbm.at[s], acc.at[pl.ds(s * STR, STR)])
    plsc.subcore_barrier()

    def body(ch, _):
        pltpu.sync_copy(ones_v, acc.at[idx_d.at[ch]], add=True)
        return ()

    lax.fori_loop(0, NCH, body, ())
    plsc.subcore_barrier()
    pltpu.sync_copy(acc.at[pl.ds(s * STR, STR)], out_hbm.at[c, s])


@functools.cache
def _deg_call_fn():
    return functools.partial(
        pl.kernel,
        out_type=jax.ShapeDtypeStruct((NC, NS, STR, DW), jnp.float32),
        mesh=plsc.VectorSubcoreMesh(core_axis_name="c", subcore_axis_name="s"),
        scratch_types=[
            pltpu.VMEM((NCH, CH), jnp.int32),
            pltpu.VMEM((CH, DW), jnp.float32),
            pltpu.VMEM_SHARED((NP, DW), jnp.float32),
        ],
        compiler_params=pltpu.CompilerParams(use_tc_tiling_on_sc=False),
    )(_deg_body)


def _deg_call(*args):
    return _deg_call_fn()(*args)


def _segsum_body(xs_hbm, srcp_hbm, dstp_hbm, zero_hbm, out_hbm,
                 idx_s, idx_d, rows0, rows1, sem0, sem1, acc):
    c = lax.axis_index("c")
    s = lax.axis_index("s")
    base = (c * NS + s) * NCH
    pltpu.sync_copy(zero_hbm.at[pl.ds(s * STR, STR)],
                    acc.at[pl.ds(s * STR, STR)])
    plsc.subcore_barrier()

    def sb_body(sb, _):
        pltpu.sync_copy(srcp_hbm.at[pl.ds(base + sb * SB, SB)], idx_s)
        pltpu.sync_copy(dstp_hbm.at[pl.ds(base + sb * SB, SB)], idx_d)
        pltpu.async_copy(xs_hbm.at[idx_s.at[0]], rows0, sem0)

        def pair_body(i, _):
            ch = i * 2
            pltpu.make_async_copy(xs_hbm.at[idx_s.at[ch]], rows0,
                                  sem0).wait()
            pltpu.async_copy(xs_hbm.at[idx_s.at[ch + 1]], rows1, sem1)
            pltpu.sync_copy(rows0, acc.at[idx_d.at[ch]], add=True)

            pltpu.make_async_copy(xs_hbm.at[idx_s.at[ch + 1]], rows1,
                                  sem1).wait()
            pltpu.async_copy(xs_hbm.at[idx_s.at[ch + 2]], rows0, sem0)
            pltpu.sync_copy(rows1, acc.at[idx_d.at[ch + 1]], add=True)
            return ()

        lax.fori_loop(0, SB // 2 - 1, pair_body, ())
        pltpu.make_async_copy(xs_hbm.at[idx_s.at[SB - 2]], rows0,
                              sem0).wait()
        pltpu.async_copy(xs_hbm.at[idx_s.at[SB - 1]], rows1, sem1)
        pltpu.sync_copy(rows0, acc.at[idx_d.at[SB - 2]], add=True)
        pltpu.make_async_copy(xs_hbm.at[idx_s.at[SB - 1]], rows1,
                              sem1).wait()
        pltpu.sync_copy(rows1, acc.at[idx_d.at[SB - 1]], add=True)
        return ()

    lax.fori_loop(0, NSB, sb_body, ())
    plsc.subcore_barrier()
    pltpu.sync_copy(acc.at[pl.ds(s * STR, STR)],
                    out_hbm.at[c, pl.ds(s * STR, STR)])


@functools.cache
def _segsum_call_fn():
    return functools.partial(
        pl.kernel,
        out_type=jax.ShapeDtypeStruct((NC, NP, D), jnp.float32),
        mesh=plsc.VectorSubcoreMesh(core_axis_name="c", subcore_axis_name="s"),
        scratch_types=[
            pltpu.VMEM((SB, CH), jnp.int32),
            pltpu.VMEM((SB, CH), jnp.int32),
            pltpu.VMEM((CH, D), jnp.float32),
            pltpu.VMEM((CH, D), jnp.float32),
            pltpu.SemaphoreType.DMA,
            pltpu.SemaphoreType.DMA,
            pltpu.VMEM_SHARED((NP, D), jnp.float32),
        ],
    )(_segsum_body)


def _segsum_call(*args):
    return _segsum_call_fn()(*args)


def _tc1a_body(rep_ref, w1_ref, xw_ref):
    xw_ref[...] = jnp.dot(rep_ref[...], w1_ref[...],
                          preferred_element_type=jnp.float32)


def _tc1a(rep, w1):
    return pl.pallas_call(
        _tc1a_body,
        grid=(N // RB,),
        in_specs=[
            pl.BlockSpec((RB, D), lambda i: (i, 0)),
            pl.BlockSpec((D, D), lambda i: (0, 0)),
        ],
        out_specs=pl.BlockSpec((RB, D), lambda i: (i, 0)),
        out_shape=jax.ShapeDtypeStruct((N, D), jnp.float32),
    )(rep, w1)


def _dinv_of(degp_ref):
    cnt = (degp_ref[0] + degp_ref[1])[:, 0:1]
    return lax.rsqrt(cnt + 2.0)


def _tc1b_body(degp_ref, xw_ref, xs_ref):
    xs_ref[...] = xw_ref[...] * _dinv_of(degp_ref)


def _tc1b(degp, xw):
    return pl.pallas_call(
        _tc1b_body,
        grid=(N // RB,),
        in_specs=[
            pl.BlockSpec((NC, RB, DW), lambda i: (0, i, 0)),
            pl.BlockSpec((RB, D), lambda i: (i, 0)),
        ],
        out_specs=pl.BlockSpec((RB, D), lambda i: (i, 0)),
        out_shape=jax.ShapeDtypeStruct((N, D), jnp.float32),
    )(degp, xw)


def _tc2_body(accp_ref, xs_ref, degp_ref, b_ref, w_ref, xs2_ref):
    acc = accp_ref[0] + accp_ref[1]
    dinv = _dinv_of(degp_ref)
    h = jnp.maximum(dinv * (acc + 2.0 * xs_ref[...]) + b_ref[...][None, :], 0.0)
    xs2_ref[...] = jnp.dot(h, w_ref[...],
                           preferred_element_type=jnp.float32) * dinv


def _tc2(accp, xs, degp, b, w):
    return pl.pallas_call(
        _tc2_body,
        grid=(N // RB,),
        in_specs=[
            pl.BlockSpec((NC, RB, D), lambda i: (0, i, 0)),
            pl.BlockSpec((RB, D), lambda i: (i, 0)),
            pl.BlockSpec((NC, RB, DW), lambda i: (0, i, 0)),
            pl.BlockSpec((D,), lambda i: (0,)),
            pl.BlockSpec((D, D), lambda i: (0, 0)),
        ],
        out_specs=pl.BlockSpec((RB, D), lambda i: (i, 0)),
        out_shape=jax.ShapeDtypeStruct((N, D), jnp.float32),
    )(accp, xs, degp, b, w)


def _tc3_body(accp_ref, xs2_ref, degp_ref, b2_ref, rep_ref, wl1a_ref,
              wl1b_ref, bl1_ref, wl2_ref, bl2_ref, u_ref,
              prob_ref, samp_ref):
    acc = accp_ref[0] + accp_ref[1]
    dinv = _dinv_of(degp_ref)
    h2 = jnp.maximum(dinv * (acc + 2.0 * xs2_ref[...]) + b2_ref[...][None, :],
                     0.0)
    z = jnp.dot(rep_ref[...], wl1a_ref[...], preferred_element_type=jnp.float32)
    z = z + jnp.dot(h2, wl1b_ref[...], preferred_element_type=jnp.float32)
    z = jnp.maximum(z + bl1_ref[...][None, :], 0.0)
    logit = jnp.dot(z, wl2_ref[...],
                    preferred_element_type=jnp.float32) + bl2_ref[0, 0]
    p = 1.0 / (1.0 + jnp.exp(-logit))
    u0 = u_ref[:, 0:1]
    u1 = u_ref[:, 1:2]
    g0 = -jnp.log(-jnp.log(u0))
    g1 = -jnp.log(-jnp.log(u1))
    prob_ref[...] = p
    samp_ref[...] = jnp.where(p + g1 > 1.0 - p + g0, 1.0, 0.0)


def _tc3(accp, xs2, degp, b2, rep, wl1a, wl1b, bl1, wl2, bl2, u):
    return pl.pallas_call(
        _tc3_body,
        grid=(N // RB,),
        in_specs=[
            pl.BlockSpec((NC, RB, D), lambda i: (0, i, 0)),
            pl.BlockSpec((RB, D), lambda i: (i, 0)),
            pl.BlockSpec((NC, RB, DW), lambda i: (0, i, 0)),
            pl.BlockSpec((D,), lambda i: (0,)),
            pl.BlockSpec((RB, D), lambda i: (i, 0)),
            pl.BlockSpec((D, 64), lambda i: (0, 0)),
            pl.BlockSpec((D, 64), lambda i: (0, 0)),
            pl.BlockSpec((64,), lambda i: (0,)),
            pl.BlockSpec((64, 1), lambda i: (0, 0)),
            pl.BlockSpec((1, 1), lambda i: (0, 0)),
            pl.BlockSpec((RB, 2), lambda i: (i, 0)),
        ],
        out_specs=[
            pl.BlockSpec((RB, 1), lambda i: (i, 0)),
            pl.BlockSpec((RB, 1), lambda i: (i, 0)),
        ],
        out_shape=[
            jax.ShapeDtypeStruct((N, 1), jnp.float32),
            jax.ShapeDtypeStruct((N, 1), jnp.float32),
        ],
    )(accp, xs2, degp, b2, rep, wl1a, wl1b, bl1, wl2, bl2, u)


def kernel(rep, edge_index, W1, b1, W2, b2, Wl1, bl1, Wl2, bl2):
    src = edge_index[0]
    dst = edge_index[1]
    srcp = jnp.concatenate([src, _PAD_SRC]).reshape(TCH, CH)
    dstp = jnp.concatenate([dst, _PAD_DST]).reshape(TCH, CH)

    ones_dw = jnp.ones((CH, DW), jnp.float32)
    zeros_dw = jnp.zeros((NS, STR, DW), jnp.float32)
    zeros_d = jnp.zeros((NP, D), jnp.float32)

    degp = _deg_call(dstp.reshape(NW, NCH, CH), ones_dw,
                     zeros_dw).reshape(NC, NP, DW)
    xw1 = _tc1a(rep, W1)
    xs1 = _tc1b(degp, xw1)
    acc1p = _segsum_call(xs1, srcp, dstp, zeros_d)
    xs2 = _tc2(acc1p, xs1, degp, b1, W2)
    acc2p = _segsum_call(xs2, srcp, dstp, zeros_d)

    u = jax.random.uniform(jax.random.key(1234), (N, 2),
                           minval=1e-8, maxval=1.0)
    wl1a = Wl1[:D]
    wl1b = Wl1[D:]
    prob, samp = _tc3(acc2p, xs2, degp, b2, rep, wl1a, wl1b, bl1, Wl2,
                      bl2.reshape(1, 1), u)
    return (prob.reshape(N), samp.reshape(N))
